# baseline (speedup 1.0000x reference)
import functools

import jax
import jax.numpy as jnp
from jax import lax
from jax.experimental import pallas as pl
from jax.experimental.pallas import tpu as pltpu
from jax.experimental.pallas import tpu_sc as plsc

_N = 1048576
_NTAB = 1000
_TAB_PAD = 1008
_LANES = 16
_SIZES = (4096, 14336, 14336)
_MAXCHUNK = max(_SIZES)


@functools.lru_cache(maxsize=None)
def _build_kernel():
    info = plsc.get_sparse_core_info()
    num_cores = info.num_cores
    num_subcores = info.num_subcores
    num_workers = num_cores * num_subcores
    per_worker = _N // num_workers
    assert sum(_SIZES) == per_worker
    n_chunks = len(_SIZES)
    offs = [sum(_SIZES[:i]) for i in range(n_chunks)]

    mesh = plsc.VectorSubcoreMesh(core_axis_name="c", subcore_axis_name="s")

    @functools.partial(
        pl.kernel,
        mesh=mesh,
        out_type=jax.ShapeDtypeStruct((_N,), jnp.float32),
        compiler_params=pltpu.CompilerParams(needs_layout_passes=False),
        scratch_types=[
            pltpu.VMEM((_TAB_PAD,), jnp.float32),
            pltpu.VMEM((_TAB_PAD,), jnp.float32),
            pltpu.VMEM((2, _MAXCHUNK), jnp.int32),
            pltpu.VMEM((2, _MAXCHUNK), jnp.float32),
            pltpu.SemaphoreType.DMA,
            pltpu.SemaphoreType.DMA,
            pltpu.SemaphoreType.DMA,
            pltpu.SemaphoreType.DMA,
        ],
    )
    def sigma_kernel(ts_hbm, ls_hbm, out_hbm, logt_v, sigt_v, idx_v, res_v,
                     in_sem0, in_sem1, out_sem0, out_sem1):
        wid = lax.axis_index("s") * num_cores + lax.axis_index("c")
        base = wid * per_worker
        in_sems = (in_sem0, in_sem1)
        out_sems = (out_sem0, out_sem1)

        def in_copy(c):
            return pltpu.make_async_copy(
                ts_hbm.at[pl.ds(base + offs[c], _SIZES[c])],
                idx_v.at[c % 2, pl.ds(0, _SIZES[c])],
                in_sems[c % 2],
            )

        def out_copy(c):
            return pltpu.make_async_copy(
                res_v.at[c % 2, pl.ds(0, _SIZES[c])],
                out_hbm.at[pl.ds(base + offs[c], _SIZES[c])],
                out_sems[c % 2],
            )

        tab_copy = pltpu.make_async_copy(
            ls_hbm, logt_v.at[pl.ds(0, _NTAB)], out_sem0)
        tab_copy.start()
        in_copy(0).start()
        tab_copy.wait()

        @plsc.parallel_loop(0, _TAB_PAD, _LANES, unroll=4)
        def exp_body(i):
            sl = pl.ds(i, _LANES)
            sigt_v[sl] = jnp.exp(logt_v[sl])

        for c in range(n_chunks):
            buf = c % 2
            if c + 1 < n_chunks:
                in_copy(c + 1).start()
            in_copy(c).wait()
            if c >= 2:
                out_copy(c - 2).wait()

            @plsc.parallel_loop(0, _SIZES[c], _LANES, unroll=16)
            def gather_body(j, _buf=buf):
                sl = pl.ds(j, _LANES)
                idx = idx_v[_buf, sl]
                res_v[_buf, sl] = plsc.load_gather(sigt_v, [idx])

            out_copy(c).start()

        out_copy(n_chunks - 2).wait()
        out_copy(n_chunks - 1).wait()

    return sigma_kernel


def kernel(timestep, log_sigmas):
    return _build_kernel()(timestep, log_sigmas)


# speedup vs baseline: 1.0099x; 1.0099x over previous
import functools

import jax
import jax.numpy as jnp
from jax import lax
from jax.experimental import pallas as pl
from jax.experimental.pallas import tpu as pltpu
from jax.experimental.pallas import tpu_sc as plsc

_N = 1048576
_NTAB = 1000
_TAB_PAD = 1008
_LANES = 16
_SIZES = (16384, 16384)
_MAXCHUNK = max(_SIZES)


@functools.lru_cache(maxsize=None)
def _build_kernel():
    info = plsc.get_sparse_core_info()
    num_cores = info.num_cores
    num_subcores = info.num_subcores
    num_workers = num_cores * num_subcores
    per_worker = _N // num_workers
    assert sum(_SIZES) == per_worker
    n_chunks = len(_SIZES)
    offs = [sum(_SIZES[:i]) for i in range(n_chunks)]

    mesh = plsc.VectorSubcoreMesh(core_axis_name="c", subcore_axis_name="s")

    @functools.partial(
        pl.kernel,
        mesh=mesh,
        out_type=jax.ShapeDtypeStruct((_N,), jnp.float32),
        compiler_params=pltpu.CompilerParams(needs_layout_passes=False),
        scratch_types=[
            pltpu.VMEM((_TAB_PAD,), jnp.float32),
            pltpu.VMEM((_TAB_PAD,), jnp.float32),
            pltpu.VMEM((2, _MAXCHUNK), jnp.int32),
            pltpu.VMEM((2, _MAXCHUNK), jnp.float32),
            pltpu.SemaphoreType.DMA,
            pltpu.SemaphoreType.DMA,
            pltpu.SemaphoreType.DMA,
            pltpu.SemaphoreType.DMA,
        ],
    )
    def sigma_kernel(ts_hbm, ls_hbm, out_hbm, logt_v, sigt_v, idx_v, res_v,
                     in_sem0, in_sem1, out_sem0, out_sem1):
        wid = lax.axis_index("s") * num_cores + lax.axis_index("c")
        base = wid * per_worker
        in_sems = (in_sem0, in_sem1)
        out_sems = (out_sem0, out_sem1)

        def in_copy(c):
            return pltpu.make_async_copy(
                ts_hbm.at[pl.ds(base + offs[c], _SIZES[c])],
                idx_v.at[c % 2, pl.ds(0, _SIZES[c])],
                in_sems[c % 2],
            )

        def out_copy(c):
            return pltpu.make_async_copy(
                res_v.at[c % 2, pl.ds(0, _SIZES[c])],
                out_hbm.at[pl.ds(base + offs[c], _SIZES[c])],
                out_sems[c % 2],
            )

        tab_copy = pltpu.make_async_copy(
            ls_hbm, logt_v.at[pl.ds(0, _NTAB)], out_sem0)
        tab_copy.start()
        in_copy(0).start()
        tab_copy.wait()

        @plsc.parallel_loop(0, _TAB_PAD, _LANES, unroll=4)
        def exp_body(i):
            sl = pl.ds(i, _LANES)
            sigt_v[sl] = jnp.exp(logt_v[sl])

        for c in range(n_chunks):
            buf = c % 2
            if c + 1 < n_chunks:
                in_copy(c + 1).start()
            in_copy(c).wait()
            if c >= 2:
                out_copy(c - 2).wait()

            @plsc.parallel_loop(0, _SIZES[c], _LANES, unroll=8)
            def gather_body(j, _buf=buf):
                sl = pl.ds(j, _LANES)
                idx = idx_v[_buf, sl]
                res_v[_buf, sl] = plsc.load_gather(sigt_v, [idx])

            out_copy(c).start()

        out_copy(n_chunks - 2).wait()
        out_copy(n_chunks - 1).wait()

    return sigma_kernel


def kernel(timestep, log_sigmas):
    return _build_kernel()(timestep, log_sigmas)
